# baseline (speedup 1.0000x reference)
import jax
import jax.numpy as jnp
from jax import lax
from jax.experimental import pallas as pl
from jax.experimental.pallas import tpu as pltpu, tpu_sc as plsc

N = 100000
C = 128
NC, NS, L = 2, 16, 16
NW = NC * NS
B0 = 32
BW = B0 * C
NBLK = N // B0
NFULL = -(-NBLK // NW)
NLONG = NBLK - NW * (NFULL - 1)
G = B0 // L
NPAIR = NFULL // 2


def _body(inp_hbm, out_hbm, idx_all, buf, sem0, sem1, semi):
    c = lax.axis_index("c")
    s = lax.axis_index("s")
    wid = s * NC + c

    cnt = jnp.where(wid < NLONG, NFULL, NFULL - 1)
    start = (NFULL - 1) * wid + jnp.minimum(wid, NLONG)
    shift = jnp.where(wid < NLONG, 0, B0)
    ibase = start * B0 - shift

    pltpu.async_copy(inp_hbm.at[pl.ds(ibase, NFULL * B0)], idx_all, semi)

    iota = lax.iota(jnp.int32, 16)
    ones = jnp.ones((16,), jnp.int32)
    zeros = jnp.zeros((16,), jnp.int32)

    def scat(j, off, val):
        for g in range(G):
            vals = idx_all[pl.ds(shift + j * B0 + g * L, L)]
            lin = off + (g * L + iota) * C + vals
            plsc.store_scatter(buf, [lin], val)

    def one_chunk(k, cc):
        for u in range(8):
            buf[pl.ds(k * 128 + u * 16, 16)] = zeros
        return cc
    lax.fori_loop(0, 2 * BW // 128, one_chunk, 0)
    pltpu.make_async_copy(
        inp_hbm.at[pl.ds(ibase, NFULL * B0)], idx_all, semi).wait()

    def half(i, j, off, sem):
        dst = out_hbm.at[pl.ds((start + j) * BW, BW)]
        src = buf.at[pl.ds(off, BW)]

        @pl.when(i >= 1)
        def _():
            pltpu.make_async_copy(src, dst, sem).wait()
            scat(j - 2, off, zeros)

        scat(j, off, ones)
        pltpu.async_copy(src, dst, sem)

    def do_pair(i, carry):
        half(i, 2 * i, 0, sem0)

        @pl.when(2 * i + 1 < cnt)
        def _():
            half(i, 2 * i + 1, BW, sem1)

        return carry

    lax.fori_loop(0, NPAIR, do_pair, 0)

    anysrc = buf.at[pl.ds(0, BW)]
    anydst = out_hbm.at[pl.ds(0, BW)]
    pltpu.make_async_copy(anysrc, anydst, sem0).wait()
    pltpu.make_async_copy(anysrc, anydst, sem1).wait()


_onehot_sc = pl.kernel(
    _body,
    out_type=jax.ShapeDtypeStruct((N * C,), jnp.int32),
    mesh=plsc.VectorSubcoreMesh(core_axis_name="c", subcore_axis_name="s"),
    scratch_types=(
        pltpu.VMEM((NFULL * B0,), jnp.int32),
        pltpu.VMEM((2 * BW,), jnp.int32),
        pltpu.SemaphoreType.DMA,
        pltpu.SemaphoreType.DMA,
        pltpu.SemaphoreType.DMA,
    ),
    compiler_params=pltpu.CompilerParams(needs_layout_passes=False),
)


def kernel(inp):
    out = _onehot_sc(inp)
    return out.reshape(N, C)


# speedup vs baseline: 1.0124x; 1.0124x over previous
import jax
import jax.numpy as jnp
from jax import lax
from jax.experimental import pallas as pl
from jax.experimental.pallas import tpu as pltpu, tpu_sc as plsc

N = 100000
C = 128
NC, NS, L = 2, 16, 16
NW = NC * NS
B0 = 32
BW = B0 * C
NBLK = N // B0
NFULL = -(-NBLK // NW)
NLONG = NBLK - NW * (NFULL - 1)
G = B0 // L


def _body(inp_hbm, out_hbm, idx_all, buf, sem0, sem1, semi):
    c = lax.axis_index("c")
    s = lax.axis_index("s")
    wid = s * NC + c

    cnt = jnp.where(wid < NLONG, NFULL, NFULL - 1)
    start = (NFULL - 1) * wid + jnp.minimum(wid, NLONG)
    shift = jnp.where(wid < NLONG, 0, B0)
    ibase = start * B0 - shift

    def idx_copy():
        return pltpu.make_async_copy(
            inp_hbm.at[pl.ds(ibase, NFULL * B0)], idx_all, semi)

    idx_copy().start()

    iota = lax.iota(jnp.int32, 16)
    ones = jnp.ones((16,), jnp.int32)
    zeros = jnp.zeros((16,), jnp.int32)

    def scat(j, off, val):
        for g in range(G):
            vals = idx_all[pl.ds(shift + j * B0 + g * L, L)]
            lin = off + (g * L + iota) * C + vals
            plsc.store_scatter(buf, [lin], val)

    def do_block(j, carry):
        @pl.when(j < cnt)
        def _():
            off = (j % 2) * BW
            dst = out_hbm.at[pl.ds((start + j) * BW, BW)]
            src = buf.at[pl.ds(off, BW)]

            @pl.when(j == 0)
            def _():
                def one_chunk(k, cc):
                    for u in range(8):
                        buf[pl.ds(k * 128 + u * 16, 16)] = zeros
                    return cc
                lax.fori_loop(0, 2 * BW // 128, one_chunk, 0)
                idx_copy().wait()

            @pl.when(j >= 2)
            def _():
                @pl.when(j % 2 == 0)
                def _():
                    pltpu.make_async_copy(src, dst, sem0).wait()

                @pl.when(j % 2 == 1)
                def _():
                    pltpu.make_async_copy(src, dst, sem1).wait()

                scat(j - 2, off, zeros)

            scat(j, off, ones)

            @pl.when(j % 2 == 0)
            def _():
                pltpu.async_copy(src, dst, sem0)

            @pl.when(j % 2 == 1)
            def _():
                pltpu.async_copy(src, dst, sem1)

        return carry

    lax.fori_loop(0, NFULL, do_block, 0)

    anysrc = buf.at[pl.ds(0, BW)]
    anydst = out_hbm.at[pl.ds(0, BW)]
    pltpu.make_async_copy(anysrc, anydst, sem0).wait()
    pltpu.make_async_copy(anysrc, anydst, sem1).wait()


_onehot_sc = pl.kernel(
    _body,
    out_type=jax.ShapeDtypeStruct((N * C,), jnp.int32),
    mesh=plsc.VectorSubcoreMesh(core_axis_name="c", subcore_axis_name="s"),
    scratch_types=(
        pltpu.VMEM((NFULL * B0,), jnp.int32),
        pltpu.VMEM((2 * BW,), jnp.int32),
        pltpu.SemaphoreType.DMA,
        pltpu.SemaphoreType.DMA,
        pltpu.SemaphoreType.DMA,
    ),
    compiler_params=pltpu.CompilerParams(
        needs_layout_passes=False, skip_device_barrier=True),
)


def kernel(inp):
    out = _onehot_sc(inp)
    return out.reshape(N, C)
